# baseline (speedup 1.0000x reference)
import functools

import jax
import jax.numpy as jnp
from jax import lax
from jax.experimental import pallas as pl
from jax.experimental.pallas import tpu as pltpu
from jax.experimental.pallas import tpu_sc as plsc

_B = 16
_C = 3
_H = 512
_W = 512


def _sc_lookup(cam_hbm, idi_hbm, dt_hbm,
               wc1_hbm, bc1_hbm, wi1_hbm, bi1_hbm,
               wc2_hbm, bc2_hbm, wi2_hbm, bi2_hbm,
               w_out, b_out,
               cam_v, idi_v, dt_v,
               wc1_v, bc1_v, wi1_v, bi1_v,
               wc2_v, bc2_v, wi2_v, bi2_v,
               w_v, b_v):
    wid = lax.axis_index("s") * 2 + lax.axis_index("c")

    @pl.when(wid == 0)
    def _():
        pltpu.sync_copy(cam_hbm, cam_v)
        pltpu.sync_copy(idi_hbm, idi_v)
        pltpu.sync_copy(dt_hbm, dt_v)
        pltpu.sync_copy(wc1_hbm, wc1_v)
        pltpu.sync_copy(bc1_hbm, bc1_v)
        pltpu.sync_copy(wi1_hbm, wi1_v)
        pltpu.sync_copy(bi1_hbm, bi1_v)
        pltpu.sync_copy(wc2_hbm, wc2_v)
        pltpu.sync_copy(bc2_hbm, bc2_v)
        pltpu.sync_copy(wi2_hbm, wi2_v)
        pltpu.sync_copy(bi2_hbm, bi2_v)

        cam = cam_v[...]
        idi = idi_v[...]
        use1 = dt_v[...] == 0
        pos0 = lax.iota(jnp.int32, _B) * _C
        for c in range(_C):
            idx_cam = cam * _C + c
            idx_id = idi * _C + c
            w1 = plsc.load_gather(wc1_v, [idx_cam]) + plsc.load_gather(wi1_v, [idx_id])
            w2 = plsc.load_gather(wc2_v, [idx_cam]) + plsc.load_gather(wi2_v, [idx_id])
            b1 = plsc.load_gather(bc1_v, [idx_cam]) + plsc.load_gather(bi1_v, [idx_id])
            b2 = plsc.load_gather(bc2_v, [idx_cam]) + plsc.load_gather(bi2_v, [idx_id])
            plsc.store_scatter(w_v, [pos0 + c], jnp.where(use1, w1, w2))
            plsc.store_scatter(b_v, [pos0 + c], jnp.where(use1, b1, b2))

        pltpu.sync_copy(w_v, w_out)
        pltpu.sync_copy(b_v, b_out)


def _affine_body(w_ref, b_ref, img_ref, out_ref):
    i = pl.program_id(0)
    out_ref[...] = img_ref[...] * w_ref[i] + b_ref[i]


def _pad_flat(t, n):
    flat = t.reshape(-1)
    return jnp.pad(flat, (0, n - flat.shape[0]))


def kernel(image, camindex, idindex, dataset_type,
           wcam1, bcam1, wident1, bident1,
           wcam2, bcam2, wident2, bident2):
    wc1 = _pad_flat(wcam1, 304)
    bc1 = _pad_flat(bcam1, 304)
    wi1 = wident1.reshape(-1)
    bi1 = bident1.reshape(-1)
    wc2 = _pad_flat(wcam2, 152)
    bc2 = _pad_flat(bcam2, 152)
    wi2 = wident2.reshape(-1)
    bi2 = bident2.reshape(-1)

    mesh = plsc.VectorSubcoreMesh(core_axis_name="c", subcore_axis_name="s")
    vec = jax.ShapeDtypeStruct((_B * _C,), jnp.float32)
    sc_fn = pl.kernel(
        _sc_lookup,
        out_type=[vec, vec],
        mesh=mesh,
        scratch_types=[
            pltpu.VMEM((_B,), jnp.int32),
            pltpu.VMEM((_B,), jnp.int32),
            pltpu.VMEM((_B,), jnp.int32),
            pltpu.VMEM((304,), jnp.float32),
            pltpu.VMEM((304,), jnp.float32),
            pltpu.VMEM((30000,), jnp.float32),
            pltpu.VMEM((30000,), jnp.float32),
            pltpu.VMEM((152,), jnp.float32),
            pltpu.VMEM((152,), jnp.float32),
            pltpu.VMEM((15000,), jnp.float32),
            pltpu.VMEM((15000,), jnp.float32),
            pltpu.VMEM((_B * _C,), jnp.float32),
            pltpu.VMEM((_B * _C,), jnp.float32),
        ],
        name="colorcal_sc_lookup",
        compiler_params=pltpu.CompilerParams(needs_layout_passes=False),
    )
    w48, b48 = sc_fn(camindex, idindex, dataset_type,
                     wc1, bc1, wi1, bi1, wc2, bc2, wi2, bi2)

    img = image.reshape(_B * _C, _H, _W)
    out = pl.pallas_call(
        _affine_body,
        grid=(_B * _C,),
        in_specs=[
            pl.BlockSpec(memory_space=pltpu.SMEM),
            pl.BlockSpec(memory_space=pltpu.SMEM),
            pl.BlockSpec((1, _H, _W), lambda i: (i, 0, 0)),
        ],
        out_specs=pl.BlockSpec((1, _H, _W), lambda i: (i, 0, 0)),
        out_shape=jax.ShapeDtypeStruct((_B * _C, _H, _W), jnp.float32),
        name="colorcal_affine",
    )(w48, b48, img)
    return out.reshape(_B, _C, _H, _W)


# speedup vs baseline: 1.1590x; 1.1590x over previous
import functools

import jax
import jax.numpy as jnp
from jax import lax
from jax.experimental import pallas as pl
from jax.experimental.pallas import tpu as pltpu
from jax.experimental.pallas import tpu_sc as plsc

_B = 16
_C = 3
_H = 512
_W = 512


def _sc_lookup(cam_hbm, idi_hbm, dt_hbm,
               wc1_hbm, bc1_hbm, wi1_hbm, bi1_hbm,
               wc2_hbm, bc2_hbm, wi2_hbm, bi2_hbm,
               w_out, b_out,
               cam_v, idi_v, dt_v,
               wc1_v, bc1_v, wi1_v, bi1_v,
               wc2_v, bc2_v, wi2_v, bi2_v,
               w_v, b_v):
    wid = lax.axis_index("s") * 2 + lax.axis_index("c")

    @pl.when(wid == 0)
    def _():
        pltpu.sync_copy(cam_hbm, cam_v)
        pltpu.sync_copy(idi_hbm, idi_v)
        pltpu.sync_copy(dt_hbm, dt_v)
        pltpu.sync_copy(wc1_hbm, wc1_v)
        pltpu.sync_copy(bc1_hbm, bc1_v)
        pltpu.sync_copy(wi1_hbm, wi1_v)
        pltpu.sync_copy(bi1_hbm, bi1_v)
        pltpu.sync_copy(wc2_hbm, wc2_v)
        pltpu.sync_copy(bc2_hbm, bc2_v)
        pltpu.sync_copy(wi2_hbm, wi2_v)
        pltpu.sync_copy(bi2_hbm, bi2_v)

        cam = cam_v[...]
        idi = idi_v[...]
        use1 = dt_v[...] == 0
        pos0 = lax.iota(jnp.int32, _B) * _C
        for c in range(_C):
            idx_cam = cam * _C + c
            idx_id = idi * _C + c
            w1 = plsc.load_gather(wc1_v, [idx_cam]) + plsc.load_gather(wi1_v, [idx_id])
            w2 = plsc.load_gather(wc2_v, [idx_cam]) + plsc.load_gather(wi2_v, [idx_id])
            b1 = plsc.load_gather(bc1_v, [idx_cam]) + plsc.load_gather(bi1_v, [idx_id])
            b2 = plsc.load_gather(bc2_v, [idx_cam]) + plsc.load_gather(bi2_v, [idx_id])
            plsc.store_scatter(w_v, [pos0 + c], jnp.where(use1, w1, w2))
            plsc.store_scatter(b_v, [pos0 + c], jnp.where(use1, b1, b2))

        pltpu.sync_copy(w_v, w_out)
        pltpu.sync_copy(b_v, b_out)


def _affine_body(w_ref, b_ref, img_ref, out_ref):
    out_ref[...] = img_ref[...] * w_ref[...] + b_ref[...]


def _pad_flat(t, n):
    flat = t.reshape(-1)
    return jnp.pad(flat, (0, n - flat.shape[0]))


def kernel(image, camindex, idindex, dataset_type,
           wcam1, bcam1, wident1, bident1,
           wcam2, bcam2, wident2, bident2):
    wc1 = _pad_flat(wcam1, 304)
    bc1 = _pad_flat(bcam1, 304)
    wi1 = wident1.reshape(-1)
    bi1 = bident1.reshape(-1)
    wc2 = _pad_flat(wcam2, 152)
    bc2 = _pad_flat(bcam2, 152)
    wi2 = wident2.reshape(-1)
    bi2 = bident2.reshape(-1)

    mesh = plsc.VectorSubcoreMesh(core_axis_name="c", subcore_axis_name="s")
    vec = jax.ShapeDtypeStruct((_B * _C,), jnp.float32)
    sc_fn = pl.kernel(
        _sc_lookup,
        out_type=[vec, vec],
        mesh=mesh,
        scratch_types=[
            pltpu.VMEM((_B,), jnp.int32),
            pltpu.VMEM((_B,), jnp.int32),
            pltpu.VMEM((_B,), jnp.int32),
            pltpu.VMEM((304,), jnp.float32),
            pltpu.VMEM((304,), jnp.float32),
            pltpu.VMEM((30000,), jnp.float32),
            pltpu.VMEM((30000,), jnp.float32),
            pltpu.VMEM((152,), jnp.float32),
            pltpu.VMEM((152,), jnp.float32),
            pltpu.VMEM((15000,), jnp.float32),
            pltpu.VMEM((15000,), jnp.float32),
            pltpu.VMEM((_B * _C,), jnp.float32),
            pltpu.VMEM((_B * _C,), jnp.float32),
        ],
        name="colorcal_sc_lookup",
        compiler_params=pltpu.CompilerParams(needs_layout_passes=False),
    )
    w48, b48 = sc_fn(camindex, idindex, dataset_type,
                     wc1, bc1, wi1, bi1, wc2, bc2, wi2, bi2)

    img = image.reshape(_B * _C, _H, _W)
    nplanes = 4
    out = pl.pallas_call(
        _affine_body,
        grid=(_B * _C // nplanes,),
        in_specs=[
            pl.BlockSpec((nplanes, 1, 1), lambda i: (i, 0, 0)),
            pl.BlockSpec((nplanes, 1, 1), lambda i: (i, 0, 0)),
            pl.BlockSpec((nplanes, _H, _W), lambda i: (i, 0, 0)),
        ],
        out_specs=pl.BlockSpec((nplanes, _H, _W), lambda i: (i, 0, 0)),
        out_shape=jax.ShapeDtypeStruct((_B * _C, _H, _W), jnp.float32),
        compiler_params=pltpu.CompilerParams(
            dimension_semantics=("arbitrary",)),
        name="colorcal_affine",
    )(w48.reshape(_B * _C, 1, 1), b48.reshape(_B * _C, 1, 1), img)
    return out.reshape(_B, _C, _H, _W)


# speedup vs baseline: 1.2175x; 1.0504x over previous
import functools

import jax
import jax.numpy as jnp
from jax import lax
from jax.experimental import pallas as pl
from jax.experimental.pallas import tpu as pltpu
from jax.experimental.pallas import tpu_sc as plsc

_B = 16
_C = 3
_H = 512
_W = 512


def _sc_lookup(cam_hbm, idi_hbm, dt_hbm,
               wc1_hbm, bc1_hbm, wi1_hbm, bi1_hbm,
               wc2_hbm, bc2_hbm, wi2_hbm, bi2_hbm,
               w_out, b_out,
               cam_v, idi_v, dt_v,
               wc1_v, bc1_v, wi1_v, bi1_v,
               wc2_v, bc2_v, wi2_v, bi2_v,
               w_v, b_v):
    wid = lax.axis_index("s") * 2 + lax.axis_index("c")

    @pl.when(wid == 0)
    def _():
        pltpu.sync_copy(cam_hbm, cam_v)
        pltpu.sync_copy(idi_hbm, idi_v)
        pltpu.sync_copy(dt_hbm, dt_v)
        pltpu.sync_copy(wc1_hbm, wc1_v)
        pltpu.sync_copy(bc1_hbm, bc1_v)
        pltpu.sync_copy(wi1_hbm, wi1_v)
        pltpu.sync_copy(bi1_hbm, bi1_v)
        pltpu.sync_copy(wc2_hbm, wc2_v)
        pltpu.sync_copy(bc2_hbm, bc2_v)
        pltpu.sync_copy(wi2_hbm, wi2_v)
        pltpu.sync_copy(bi2_hbm, bi2_v)

        cam = cam_v[...]
        idi = idi_v[...]
        use1 = dt_v[...] == 0
        pos0 = lax.iota(jnp.int32, _B) * _C
        for c in range(_C):
            idx_cam = cam * _C + c
            idx_id = idi * _C + c
            w1 = plsc.load_gather(wc1_v, [idx_cam]) + plsc.load_gather(wi1_v, [idx_id])
            w2 = plsc.load_gather(wc2_v, [idx_cam]) + plsc.load_gather(wi2_v, [idx_id])
            b1 = plsc.load_gather(bc1_v, [idx_cam]) + plsc.load_gather(bi1_v, [idx_id])
            b2 = plsc.load_gather(bc2_v, [idx_cam]) + plsc.load_gather(bi2_v, [idx_id])
            plsc.store_scatter(w_v, [pos0 + c], jnp.where(use1, w1, w2))
            plsc.store_scatter(b_v, [pos0 + c], jnp.where(use1, b1, b2))

        pltpu.sync_copy(w_v, w_out)
        pltpu.sync_copy(b_v, b_out)


_P = 4
_NBUF = 4
_NCHUNK = (_B * _C) // _P


def _affine_body(w_ref, b_ref, img_ref, out_ref, buf_in, buf_out, sem_in, sem_out):
    def in_copy(k, s):
        return pltpu.make_async_copy(
            img_ref.at[pl.ds(k * _P, _P)], buf_in.at[s], sem_in.at[s])

    def out_copy(k, s):
        return pltpu.make_async_copy(
            buf_out.at[s], out_ref.at[pl.ds(k * _P, _P)], sem_out.at[s])

    for s in range(_NBUF):
        in_copy(s, s).start()
    for k in range(_NCHUNK):
        s = k % _NBUF
        in_copy(k, s).wait()
        if k >= _NBUF:
            out_copy(k - _NBUF, s).wait()
        w = w_ref[pl.ds(k * _P, _P)].reshape(_P, 1, 1)
        b = b_ref[pl.ds(k * _P, _P)].reshape(_P, 1, 1)
        buf_out[s] = buf_in[s] * w + b
        out_copy(k, s).start()
        if k + _NBUF < _NCHUNK:
            in_copy(k + _NBUF, s).start()
    for k in range(_NCHUNK - _NBUF, _NCHUNK):
        out_copy(k, k % _NBUF).wait()


def _pad_flat(t, n):
    flat = t.reshape(-1)
    return jnp.pad(flat, (0, n - flat.shape[0]))


def kernel(image, camindex, idindex, dataset_type,
           wcam1, bcam1, wident1, bident1,
           wcam2, bcam2, wident2, bident2):
    wc1 = _pad_flat(wcam1, 304)
    bc1 = _pad_flat(bcam1, 304)
    wi1 = wident1.reshape(-1)
    bi1 = bident1.reshape(-1)
    wc2 = _pad_flat(wcam2, 152)
    bc2 = _pad_flat(bcam2, 152)
    wi2 = wident2.reshape(-1)
    bi2 = bident2.reshape(-1)

    mesh = plsc.VectorSubcoreMesh(core_axis_name="c", subcore_axis_name="s")
    vec = jax.ShapeDtypeStruct((_B * _C,), jnp.float32)
    sc_fn = pl.kernel(
        _sc_lookup,
        out_type=[vec, vec],
        mesh=mesh,
        scratch_types=[
            pltpu.VMEM((_B,), jnp.int32),
            pltpu.VMEM((_B,), jnp.int32),
            pltpu.VMEM((_B,), jnp.int32),
            pltpu.VMEM((304,), jnp.float32),
            pltpu.VMEM((304,), jnp.float32),
            pltpu.VMEM((30000,), jnp.float32),
            pltpu.VMEM((30000,), jnp.float32),
            pltpu.VMEM((152,), jnp.float32),
            pltpu.VMEM((152,), jnp.float32),
            pltpu.VMEM((15000,), jnp.float32),
            pltpu.VMEM((15000,), jnp.float32),
            pltpu.VMEM((_B * _C,), jnp.float32),
            pltpu.VMEM((_B * _C,), jnp.float32),
        ],
        name="colorcal_sc_lookup",
        compiler_params=pltpu.CompilerParams(needs_layout_passes=False),
    )
    w48, b48 = sc_fn(camindex, idindex, dataset_type,
                     wc1, bc1, wi1, bi1, wc2, bc2, wi2, bi2)

    img = image.reshape(_B * _C, _H, _W)
    out = pl.pallas_call(
        _affine_body,
        in_specs=[
            pl.BlockSpec(memory_space=pltpu.VMEM),
            pl.BlockSpec(memory_space=pltpu.VMEM),
            pl.BlockSpec(memory_space=pl.ANY),
        ],
        out_specs=pl.BlockSpec(memory_space=pl.ANY),
        out_shape=jax.ShapeDtypeStruct((_B * _C, _H, _W), jnp.float32),
        scratch_shapes=[
            pltpu.VMEM((_NBUF, _P, _H, _W), jnp.float32),
            pltpu.VMEM((_NBUF, _P, _H, _W), jnp.float32),
            pltpu.SemaphoreType.DMA((_NBUF,)),
            pltpu.SemaphoreType.DMA((_NBUF,)),
        ],
        name="colorcal_affine",
    )(w48, b48, img)
    return out.reshape(_B, _C, _H, _W)


# speedup vs baseline: 1.2918x; 1.0611x over previous
import jax
import jax.numpy as jnp
from jax import lax
from jax.experimental import pallas as pl
from jax.experimental.pallas import tpu as pltpu
from jax.experimental.pallas import tpu_sc as plsc

_B = 16
_C = 3
_H = 512
_W = 512

_P = 4
_NBUF = 4
_NCHUNK = (_B * _C) // _P


def _sc_lookup(cam_hbm, idi_hbm, dt_hbm,
               wc1_hbm, bc1_hbm, wi1_hbm, bi1_hbm,
               wc2_hbm, bc2_hbm, wi2_hbm, bi2_hbm,
               w_out, b_out,
               cam_v, idi_v, dt_v,
               wc1_v, bc1_v, wi1_v, bi1_v,
               wc2_v, bc2_v, wi2_v, bi2_v,
               w_v, b_v, sem):
    wid = lax.axis_index("s") * 2 + lax.axis_index("c")

    @pl.when(wid == 0)
    def _():
        copies = [
            pltpu.async_copy(src, dst, sem)
            for src, dst in (
                (cam_hbm, cam_v), (idi_hbm, idi_v), (dt_hbm, dt_v),
                (wc1_hbm, wc1_v), (bc1_hbm, bc1_v),
                (wi1_hbm, wi1_v), (bi1_hbm, bi1_v),
                (wc2_hbm, wc2_v), (bc2_hbm, bc2_v),
                (wi2_hbm, wi2_v), (bi2_hbm, bi2_v),
            )
        ]
        for cp in copies:
            cp.wait()

        cam = cam_v[...]
        idi = idi_v[...]
        use1 = dt_v[...] == 0
        pos0 = lax.iota(jnp.int32, _B) * _C
        for c in range(_C):
            ic = cam * _C + c
            ii = idi * _C + c
            w1 = plsc.load_gather(wc1_v, [ic]) + plsc.load_gather(wi1_v, [ii])
            w2 = plsc.load_gather(wc2_v, [ic]) + plsc.load_gather(wi2_v, [ii])
            b1 = plsc.load_gather(bc1_v, [ic]) + plsc.load_gather(bi1_v, [ii])
            b2 = plsc.load_gather(bc2_v, [ic]) + plsc.load_gather(bi2_v, [ii])
            plsc.store_scatter(w_v, [pos0 + c], jnp.where(use1, w1, w2))
            plsc.store_scatter(b_v, [pos0 + c], jnp.where(use1, b1, b2))

        pltpu.sync_copy(w_v, w_out)
        pltpu.sync_copy(b_v, b_out)


def _affine_body(w_ref, b_ref, img_ref, out_ref, buf_in, buf_out, sem_in, sem_out):
    def in_copy(k, s):
        return pltpu.make_async_copy(
            img_ref.at[pl.ds(k * _P, _P)], buf_in.at[s], sem_in.at[s])

    def out_copy(k, s):
        return pltpu.make_async_copy(
            buf_out.at[s], out_ref.at[pl.ds(k * _P, _P)], sem_out.at[s])

    for s in range(_NBUF):
        in_copy(s, s).start()
    for k in range(_NCHUNK):
        s = k % _NBUF
        in_copy(k, s).wait()
        if k >= _NBUF:
            out_copy(k - _NBUF, s).wait()
        w = w_ref[pl.ds(k * _P, _P)].reshape(_P, 1, 1)
        b = b_ref[pl.ds(k * _P, _P)].reshape(_P, 1, 1)
        buf_out[s] = buf_in[s] * w + b
        out_copy(k, s).start()
        if k + _NBUF < _NCHUNK:
            in_copy(k + _NBUF, s).start()
    for k in range(_NCHUNK - _NBUF, _NCHUNK):
        out_copy(k, k % _NBUF).wait()


def kernel(image, camindex, idindex, dataset_type,
           wcam1, bcam1, wident1, bident1,
           wcam2, bcam2, wident2, bident2):
    mesh = plsc.VectorSubcoreMesh(core_axis_name="c", subcore_axis_name="s")
    vec = jax.ShapeDtypeStruct((_B * _C,), jnp.float32)
    sc_fn = pl.kernel(
        _sc_lookup,
        out_type=[vec, vec],
        mesh=mesh,
        scratch_types=[
            pltpu.VMEM((_B,), jnp.int32),
            pltpu.VMEM((_B,), jnp.int32),
            pltpu.VMEM((_B,), jnp.int32),
            pltpu.VMEM((100 * _C,), jnp.float32),
            pltpu.VMEM((100 * _C,), jnp.float32),
            pltpu.VMEM((10000 * _C,), jnp.float32),
            pltpu.VMEM((10000 * _C,), jnp.float32),
            pltpu.VMEM((50 * _C,), jnp.float32),
            pltpu.VMEM((50 * _C,), jnp.float32),
            pltpu.VMEM((5000 * _C,), jnp.float32),
            pltpu.VMEM((5000 * _C,), jnp.float32),
            pltpu.VMEM((_B * _C,), jnp.float32),
            pltpu.VMEM((_B * _C,), jnp.float32),
            pltpu.SemaphoreType.DMA,
        ],
        name="colorcal_sc_lookup",
        compiler_params=pltpu.CompilerParams(needs_layout_passes=False),
    )
    w48, b48 = sc_fn(camindex, idindex, dataset_type,
                     wcam1.reshape(-1), bcam1.reshape(-1),
                     wident1.reshape(-1), bident1.reshape(-1),
                     wcam2.reshape(-1), bcam2.reshape(-1),
                     wident2.reshape(-1), bident2.reshape(-1))

    img = image.reshape(_B * _C, _H, _W)
    out = pl.pallas_call(
        _affine_body,
        in_specs=[
            pl.BlockSpec(memory_space=pltpu.VMEM),
            pl.BlockSpec(memory_space=pltpu.VMEM),
            pl.BlockSpec(memory_space=pl.ANY),
        ],
        out_specs=pl.BlockSpec(memory_space=pl.ANY),
        out_shape=jax.ShapeDtypeStruct((_B * _C, _H, _W), jnp.float32),
        scratch_shapes=[
            pltpu.VMEM((_NBUF, _P, _H, _W), jnp.float32),
            pltpu.VMEM((_NBUF, _P, _H, _W), jnp.float32),
            pltpu.SemaphoreType.DMA((_NBUF,)),
            pltpu.SemaphoreType.DMA((_NBUF,)),
        ],
        name="colorcal_affine",
    )(w48, b48, img)
    return out.reshape(_B, _C, _H, _W)


# speedup vs baseline: 1.3348x; 1.0333x over previous
import jax
import jax.numpy as jnp
from jax import lax
from jax.experimental import pallas as pl
from jax.experimental.pallas import tpu as pltpu
from jax.experimental.pallas import tpu_sc as plsc

_B = 16
_C = 3
_H = 512
_W = 512

_OFF_CAM1 = 0
_OFF_ID1 = 300
_OFF_CAM2 = 30300
_OFF_ID2 = 30450
_TBL = 45456

_P = 4
_NBUF = 4
_NCHUNK = (_B * _C) // _P


def _sc_lookup(cam_hbm, idi_hbm, dt_hbm, tw_hbm, tb_hbm,
               w_out, b_out,
               cam_v, idi_v, dt_v, tw_v, tb_v, w_v, b_v, sem):
    wid = lax.axis_index("s") * 2 + lax.axis_index("c")

    @pl.when(wid == 0)
    def _():
        copies = [
            pltpu.async_copy(src, dst, sem)
            for src, dst in (
                (cam_hbm, cam_v), (idi_hbm, idi_v), (dt_hbm, dt_v),
                (tw_hbm, tw_v), (tb_hbm, tb_v),
            )
        ]
        for cp in copies:
            cp.wait()

        use1 = dt_v[...] == 0
        icam = jnp.where(use1, _OFF_CAM1, _OFF_CAM2) + cam_v[...] * _C
        iid = jnp.where(use1, _OFF_ID1, _OFF_ID2) + idi_v[...] * _C
        pos0 = lax.iota(jnp.int32, _B) * _C
        for c in range(_C):
            ic = icam + c
            ii = iid + c
            plsc.store_scatter(
                w_v, [pos0 + c],
                plsc.load_gather(tw_v, [ic]) + plsc.load_gather(tw_v, [ii]))
            plsc.store_scatter(
                b_v, [pos0 + c],
                plsc.load_gather(tb_v, [ic]) + plsc.load_gather(tb_v, [ii]))

        pltpu.sync_copy(w_v, w_out)
        pltpu.sync_copy(b_v, b_out)


def _affine_body(w_ref, b_ref, img_ref, out_ref, buf_in, buf_out, sem_in, sem_out):
    def in_copy(k, s):
        return pltpu.make_async_copy(
            img_ref.at[pl.ds(k * _P, _P)], buf_in.at[s], sem_in.at[s])

    def out_copy(k, s):
        return pltpu.make_async_copy(
            buf_out.at[s], out_ref.at[pl.ds(k * _P, _P)], sem_out.at[s])

    for s in range(_NBUF):
        in_copy(s, s).start()
    for k in range(_NCHUNK):
        s = k % _NBUF
        in_copy(k, s).wait()
        if k >= _NBUF:
            out_copy(k - _NBUF, s).wait()
        w = w_ref[pl.ds(k * _P, _P)].reshape(_P, 1, 1)
        b = b_ref[pl.ds(k * _P, _P)].reshape(_P, 1, 1)
        buf_out[s] = buf_in[s] * w + b
        out_copy(k, s).start()
        if k + _NBUF < _NCHUNK:
            in_copy(k + _NBUF, s).start()
    for k in range(_NCHUNK - _NBUF, _NCHUNK):
        out_copy(k, k % _NBUF).wait()


def kernel(image, camindex, idindex, dataset_type,
           wcam1, bcam1, wident1, bident1,
           wcam2, bcam2, wident2, bident2):
    pad = jnp.zeros((6,), jnp.float32)
    tw = jnp.concatenate([wcam1.reshape(-1), wident1.reshape(-1),
                          wcam2.reshape(-1), wident2.reshape(-1), pad])
    tb = jnp.concatenate([bcam1.reshape(-1), bident1.reshape(-1),
                          bcam2.reshape(-1), bident2.reshape(-1), pad])

    mesh = plsc.VectorSubcoreMesh(core_axis_name="c", subcore_axis_name="s")
    vec = jax.ShapeDtypeStruct((_B * _C,), jnp.float32)
    sc_fn = pl.kernel(
        _sc_lookup,
        out_type=[vec, vec],
        mesh=mesh,
        scratch_types=[
            pltpu.VMEM((_B,), jnp.int32),
            pltpu.VMEM((_B,), jnp.int32),
            pltpu.VMEM((_B,), jnp.int32),
            pltpu.VMEM((_TBL,), jnp.float32),
            pltpu.VMEM((_TBL,), jnp.float32),
            pltpu.VMEM((_B * _C,), jnp.float32),
            pltpu.VMEM((_B * _C,), jnp.float32),
            pltpu.SemaphoreType.DMA,
        ],
        name="colorcal_sc_lookup",
        compiler_params=pltpu.CompilerParams(needs_layout_passes=False),
    )
    w48, b48 = sc_fn(camindex, idindex, dataset_type, tw, tb)

    img = image.reshape(_B * _C, _H, _W)
    out = pl.pallas_call(
        _affine_body,
        in_specs=[
            pl.BlockSpec(memory_space=pltpu.VMEM),
            pl.BlockSpec(memory_space=pltpu.VMEM),
            pl.BlockSpec(memory_space=pl.ANY),
        ],
        out_specs=pl.BlockSpec(memory_space=pl.ANY),
        out_shape=jax.ShapeDtypeStruct((_B * _C, _H, _W), jnp.float32),
        scratch_shapes=[
            pltpu.VMEM((_NBUF, _P, _H, _W), jnp.float32),
            pltpu.VMEM((_NBUF, _P, _H, _W), jnp.float32),
            pltpu.SemaphoreType.DMA((_NBUF,)),
            pltpu.SemaphoreType.DMA((_NBUF,)),
        ],
        name="colorcal_affine",
    )(w48, b48, img)
    return out.reshape(_B, _C, _H, _W)


# speedup vs baseline: 1.5122x; 1.1329x over previous
import jax
import jax.numpy as jnp
from jax import lax
from jax.experimental import pallas as pl
from jax.experimental.pallas import tpu as pltpu
from jax.experimental.pallas import tpu_sc as plsc

_B = 16
_C = 3
_H = 512
_W = 512

_P = 4
_NBUF = 4
_NCHUNK = (_B * _C) // _P


def _sc_lookup(cam_hbm, idi_hbm, dt_hbm,
               wc1_hbm, bc1_hbm, wi1_hbm, bi1_hbm,
               wc2_hbm, bc2_hbm, wi2_hbm, bi2_hbm,
               w_out, b_out,
               cam_v, idi_v, dt_v,
               g_cam, g_id, r_v, sem):
    wid = lax.axis_index("s") * 2 + lax.axis_index("c")

    def lookup(cam_tbl1, id_tbl1, cam_tbl2, id_tbl2, out_ref):
        cp = [pltpu.async_copy(src, dst, sem)
              for src, dst in ((cam_hbm, cam_v), (idi_hbm, idi_v),
                               (dt_hbm, dt_v))]
        for c in cp:
            c.wait()
        cam = cam_v[...]
        idi = idi_v[...]
        dt = dt_v[...]
        for i in range(_B):
            cam_i = cam[i]
            idi_i = idi[i]
            use1 = dt[i] == 0

            @pl.when(use1)
            def _():
                pltpu.async_copy(cam_tbl1.at[pl.ds(cam_i, 1)],
                                 g_cam.at[pl.ds(i, 1)], sem)
                pltpu.async_copy(id_tbl1.at[pl.ds(idi_i, 1)],
                                 g_id.at[pl.ds(i, 1)], sem)

            @pl.when(jnp.logical_not(use1))
            def _():
                pltpu.async_copy(cam_tbl2.at[pl.ds(cam_i, 1)],
                                 g_cam.at[pl.ds(i, 1)], sem)
                pltpu.async_copy(id_tbl2.at[pl.ds(idi_i, 1)],
                                 g_id.at[pl.ds(i, 1)], sem)
        for i in range(_B):
            pltpu.make_async_copy(cam_tbl1.at[pl.ds(0, 1)],
                                  g_cam.at[pl.ds(i, 1)], sem).wait()
            pltpu.make_async_copy(id_tbl1.at[pl.ds(0, 1)],
                                  g_id.at[pl.ds(i, 1)], sem).wait()

        samp = lax.iota(jnp.int32, _B)
        for c in range(_C):
            cvec = jnp.full((_B,), c, jnp.int32)
            v = (plsc.load_gather(g_cam, [samp, cvec])
                 + plsc.load_gather(g_id, [samp, cvec]))
            plsc.store_scatter(r_v, [samp * _C + c], v)
        pltpu.sync_copy(r_v, out_ref)

    @pl.when(wid == 0)
    def _():
        lookup(wc1_hbm, wi1_hbm, wc2_hbm, wi2_hbm, w_out)

    @pl.when(wid == 1)
    def _():
        lookup(bc1_hbm, bi1_hbm, bc2_hbm, bi2_hbm, b_out)


def _affine_body(w_ref, b_ref, img_ref, out_ref, buf_in, buf_out, sem_in, sem_out):
    def in_copy(k, s):
        return pltpu.make_async_copy(
            img_ref.at[pl.ds(k * _P, _P)], buf_in.at[s], sem_in.at[s])

    def out_copy(k, s):
        return pltpu.make_async_copy(
            buf_out.at[s], out_ref.at[pl.ds(k * _P, _P)], sem_out.at[s])

    for s in range(_NBUF):
        in_copy(s, s).start()
    for k in range(_NCHUNK):
        s = k % _NBUF
        in_copy(k, s).wait()
        if k >= _NBUF:
            out_copy(k - _NBUF, s).wait()
        w = w_ref[pl.ds(k * _P, _P)].reshape(_P, 1, 1)
        b = b_ref[pl.ds(k * _P, _P)].reshape(_P, 1, 1)
        buf_out[s] = buf_in[s] * w + b
        out_copy(k, s).start()
        if k + _NBUF < _NCHUNK:
            in_copy(k + _NBUF, s).start()
    for k in range(_NCHUNK - _NBUF, _NCHUNK):
        out_copy(k, k % _NBUF).wait()


def kernel(image, camindex, idindex, dataset_type,
           wcam1, bcam1, wident1, bident1,
           wcam2, bcam2, wident2, bident2):
    mesh = plsc.VectorSubcoreMesh(core_axis_name="c", subcore_axis_name="s")
    vec = jax.ShapeDtypeStruct((_B * _C,), jnp.float32)
    sc_fn = pl.kernel(
        _sc_lookup,
        out_type=[vec, vec],
        mesh=mesh,
        scratch_types=[
            pltpu.VMEM((_B,), jnp.int32),
            pltpu.VMEM((_B,), jnp.int32),
            pltpu.VMEM((_B,), jnp.int32),
            pltpu.VMEM((_B, _C), jnp.float32),
            pltpu.VMEM((_B, _C), jnp.float32),
            pltpu.VMEM((_B * _C,), jnp.float32),
            pltpu.SemaphoreType.DMA,
        ],
        name="colorcal_sc_lookup",
        compiler_params=pltpu.CompilerParams(needs_layout_passes=False,
                                             use_tc_tiling_on_sc=True),
    )
    w48, b48 = sc_fn(camindex, idindex, dataset_type,
                     wcam1, bcam1, wident1, bident1,
                     wcam2, bcam2, wident2, bident2)

    img = image.reshape(_B * _C, _H, _W)
    out = pl.pallas_call(
        _affine_body,
        in_specs=[
            pl.BlockSpec(memory_space=pltpu.VMEM),
            pl.BlockSpec(memory_space=pltpu.VMEM),
            pl.BlockSpec(memory_space=pl.ANY),
        ],
        out_specs=pl.BlockSpec(memory_space=pl.ANY),
        out_shape=jax.ShapeDtypeStruct((_B * _C, _H, _W), jnp.float32),
        scratch_shapes=[
            pltpu.VMEM((_NBUF, _P, _H, _W), jnp.float32),
            pltpu.VMEM((_NBUF, _P, _H, _W), jnp.float32),
            pltpu.SemaphoreType.DMA((_NBUF,)),
            pltpu.SemaphoreType.DMA((_NBUF,)),
        ],
        name="colorcal_affine",
    )(w48, b48, img)
    return out.reshape(_B, _C, _H, _W)
